# Initial kernel scaffold; baseline (speedup 1.0000x reference)
import jax
import jax.numpy as jnp
from jax.experimental import pallas as pl


def kernel(tokens, wte_weight, learned_embedding):
    raise NotImplementedError("write your pallas kernel here")


# speedup vs baseline: 6.1285x; 6.1285x over previous
import jax
import jax.numpy as jnp
from jax import lax
from jax.experimental import pallas as pl
from jax.experimental.pallas import tpu as pltpu
from jax.experimental.pallas import tpu_sc as plsc

VOCAB = 100000
D_MODEL = 1024
BATCH = 4
SEQ = 2048
N_TOKENS = 20

_NC = 2
_NS = 16
_NW = _NC * _NS
_ROWS = BATCH * SEQ
_RPW = _ROWS // _NW
_CH = 32
_NCHUNK = _RPW // _CH
_WPB = SEQ // _RPW
_PPAD = 32


def _body(idx_hbm, wte_hbm, learned_hbm, pidx_hbm, out_hbm,
          idx_v, rows_v, learned_v, pidx_v, sem):
    wid = lax.axis_index("s") * _NC + lax.axis_index("c")
    base = wid * _RPW
    pltpu.sync_copy(idx_hbm.at[pl.ds(base, _RPW)], idx_v)
    for j in range(_NCHUNK):
        pltpu.async_copy(
            wte_hbm.at[idx_v.at[pl.ds(j * _CH, _CH)]], rows_v, sem
        ).wait()
        pltpu.sync_copy(rows_v, out_hbm.at[pl.ds(base + j * _CH, _CH)])

    @pl.when(wid % _WPB == 0)
    def _():
        b = wid // _WPB
        pltpu.sync_copy(pidx_hbm.at[pl.ds(b * _PPAD, _PPAD)], pidx_v)
        pltpu.sync_copy(learned_hbm, learned_v)
        pltpu.async_copy(learned_v, out_hbm.at[pidx_v], sem).wait()


@jax.jit
def _soft_embedding(tokens, wte_weight, learned_embedding):
    idx = tokens.reshape(_ROWS).astype(jnp.int32)
    learned_pad = jnp.concatenate(
        [learned_embedding, learned_embedding[: _PPAD - N_TOKENS]], axis=0
    )
    t = jnp.arange(_PPAD, dtype=jnp.int32) % N_TOKENS
    pidx = (jnp.arange(BATCH, dtype=jnp.int32)[:, None] * SEQ + 1 + t[None, :]
            ).reshape(BATCH * _PPAD)
    mesh = plsc.VectorSubcoreMesh(core_axis_name="c", subcore_axis_name="s")
    out = pl.kernel(
        _body,
        out_type=jax.ShapeDtypeStruct((_ROWS, D_MODEL), jnp.float32),
        mesh=mesh,
        scratch_types=[
            pltpu.VMEM((_RPW,), jnp.int32),
            pltpu.VMEM((_CH, D_MODEL), jnp.float32),
            pltpu.VMEM((_PPAD, D_MODEL), jnp.float32),
            pltpu.VMEM((_PPAD,), jnp.int32),
            pltpu.SemaphoreType.DMA,
        ],
    )(idx, wte_weight, learned_pad, pidx)
    return out.reshape(BATCH, SEQ, D_MODEL)


def kernel(tokens, wte_weight, learned_embedding):
    return _soft_embedding(tokens, wte_weight, learned_embedding)


# speedup vs baseline: 6.7444x; 1.1005x over previous
import jax
import jax.numpy as jnp
from jax import lax
from jax.experimental import pallas as pl
from jax.experimental.pallas import tpu as pltpu
from jax.experimental.pallas import tpu_sc as plsc

VOCAB = 100000
D_MODEL = 1024
BATCH = 4
SEQ = 2048
N_TOKENS = 20

_NC = 2
_NS = 16
_NW = _NC * _NS
_ROWS = BATCH * SEQ
_RPW = _ROWS // _NW
_CH = 32
_NCHUNK = _RPW // _CH
_WPB = SEQ // _RPW
_PPAD = 32


def _body(idx_hbm, wte_hbm, learned_hbm, pidx_hbm, out_hbm,
          idx_v, rows0_v, rows1_v, pidx_v, gsem0, gsem1, ssem0, ssem1):
    wid = lax.axis_index("s") * _NC + lax.axis_index("c")
    base = wid * _RPW
    pltpu.sync_copy(idx_hbm.at[pl.ds(base, _RPW)], idx_v)

    bufs = (rows0_v, rows1_v)
    gsems = (gsem0, gsem1)
    ssems = (ssem0, ssem1)

    def gather(j):
        return pltpu.async_copy(
            wte_hbm.at[idx_v.at[pl.ds(j * _CH, _CH)]], bufs[j % 2],
            gsems[j % 2])

    def scatter(j):
        return pltpu.async_copy(
            bufs[j % 2], out_hbm.at[pl.ds(base + j * _CH, _CH)],
            ssems[j % 2])

    g = {0: gather(0), 1: gather(1)}
    s = {}
    for j in range(_NCHUNK):
        g[j].wait()
        s[j] = scatter(j)
        if j + 2 < _NCHUNK:
            s[j].wait()
            g[j + 2] = gather(j + 2)
    s[_NCHUNK - 2].wait()
    s[_NCHUNK - 1].wait()

    @pl.when(wid % _WPB == 0)
    def _():
        b = wid // _WPB
        pltpu.sync_copy(pidx_hbm.at[pl.ds(b * _PPAD, _PPAD)], pidx_v)
        pltpu.sync_copy(learned_hbm, rows0_v)
        pltpu.async_copy(rows0_v, out_hbm.at[pidx_v], gsem0).wait()


@jax.jit
def _soft_embedding(tokens, wte_weight, learned_embedding):
    idx = tokens.reshape(_ROWS).astype(jnp.int32)
    learned_pad = jnp.concatenate(
        [learned_embedding, learned_embedding[: _PPAD - N_TOKENS]], axis=0
    )
    t = jnp.arange(_PPAD, dtype=jnp.int32) % N_TOKENS
    pidx = (jnp.arange(BATCH, dtype=jnp.int32)[:, None] * SEQ + 1 + t[None, :]
            ).reshape(BATCH * _PPAD)
    mesh = plsc.VectorSubcoreMesh(core_axis_name="c", subcore_axis_name="s")
    out = pl.kernel(
        _body,
        out_type=jax.ShapeDtypeStruct((_ROWS, D_MODEL), jnp.float32),
        mesh=mesh,
        scratch_types=[
            pltpu.VMEM((_RPW,), jnp.int32),
            pltpu.VMEM((_CH, D_MODEL), jnp.float32),
            pltpu.VMEM((_CH, D_MODEL), jnp.float32),
            pltpu.VMEM((_PPAD,), jnp.int32),
            pltpu.SemaphoreType.DMA,
            pltpu.SemaphoreType.DMA,
            pltpu.SemaphoreType.DMA,
            pltpu.SemaphoreType.DMA,
        ],
    )(idx, wte_weight, learned_pad, pidx)
    return out.reshape(BATCH, SEQ, D_MODEL)


def kernel(tokens, wte_weight, learned_embedding):
    return _soft_embedding(tokens, wte_weight, learned_embedding)


# speedup vs baseline: 6.8497x; 1.0156x over previous
import jax
import jax.numpy as jnp
from jax import lax
from jax.experimental import pallas as pl
from jax.experimental.pallas import tpu as pltpu
from jax.experimental.pallas import tpu_sc as plsc

VOCAB = 100000
D_MODEL = 1024
BATCH = 4
SEQ = 2048
N_TOKENS = 20

_NC = 2
_NS = 16
_NW = _NC * _NS
_ROWS = BATCH * SEQ
_RPW = _ROWS // _NW
_CH = 32
_NCHUNK = _RPW // _CH
_WPB = SEQ // _RPW
_PPAD = 32
_NBUF = 3


def _body(idx_hbm, wte_hbm, learned_hbm, pidx_hbm, out_hbm,
          idx_v, rows0_v, rows1_v, rows2_v, pidx_v,
          gsem0, gsem1, gsem2, ssem0, ssem1, ssem2):
    wid = lax.axis_index("s") * _NC + lax.axis_index("c")
    base = wid * _RPW
    pltpu.sync_copy(idx_hbm.at[pl.ds(base, _RPW)], idx_v)

    bufs = (rows0_v, rows1_v, rows2_v)
    gsems = (gsem0, gsem1, gsem2)
    ssems = (ssem0, ssem1, ssem2)

    def gather(j):
        return pltpu.async_copy(
            wte_hbm.at[idx_v.at[pl.ds(j * _CH, _CH)]], bufs[j % _NBUF],
            gsems[j % _NBUF])

    def scatter(j):
        return pltpu.async_copy(
            bufs[j % _NBUF], out_hbm.at[pl.ds(base + j * _CH, _CH)],
            ssems[j % _NBUF])

    g = {j: gather(j) for j in range(_NBUF)}
    s = {}
    for j in range(_NCHUNK):
        g[j].wait()
        s[j] = scatter(j)
        if j + _NBUF < _NCHUNK:
            s[j].wait()
            g[j + _NBUF] = gather(j + _NBUF)
    for j in range(_NCHUNK - _NBUF, _NCHUNK):
        s[j].wait()

    @pl.when(wid % _WPB == 0)
    def _():
        b = wid // _WPB
        pltpu.sync_copy(pidx_hbm.at[pl.ds(b * _PPAD, _PPAD)], pidx_v)
        pltpu.sync_copy(learned_hbm, rows0_v)
        pltpu.async_copy(rows0_v, out_hbm.at[pidx_v], gsem0).wait()


@jax.jit
def _soft_embedding(tokens, wte_weight, learned_embedding):
    idx = tokens.reshape(_ROWS).astype(jnp.int32)
    learned_pad = jnp.concatenate(
        [learned_embedding, learned_embedding[: _PPAD - N_TOKENS]], axis=0
    )
    t = jnp.arange(_PPAD, dtype=jnp.int32) % N_TOKENS
    pidx = (jnp.arange(BATCH, dtype=jnp.int32)[:, None] * SEQ + 1 + t[None, :]
            ).reshape(BATCH * _PPAD)
    mesh = plsc.VectorSubcoreMesh(core_axis_name="c", subcore_axis_name="s")
    out = pl.kernel(
        _body,
        out_type=jax.ShapeDtypeStruct((_ROWS, D_MODEL), jnp.float32),
        mesh=mesh,
        scratch_types=[
            pltpu.VMEM((_RPW,), jnp.int32),
            pltpu.VMEM((_CH, D_MODEL), jnp.float32),
            pltpu.VMEM((_CH, D_MODEL), jnp.float32),
            pltpu.VMEM((_CH, D_MODEL), jnp.float32),
            pltpu.VMEM((_PPAD,), jnp.int32),
            pltpu.SemaphoreType.DMA,
            pltpu.SemaphoreType.DMA,
            pltpu.SemaphoreType.DMA,
            pltpu.SemaphoreType.DMA,
            pltpu.SemaphoreType.DMA,
            pltpu.SemaphoreType.DMA,
        ],
    )(idx, wte_weight, learned_pad, pidx)
    return out.reshape(BATCH, SEQ, D_MODEL)


def kernel(tokens, wte_weight, learned_embedding):
    return _soft_embedding(tokens, wte_weight, learned_embedding)


# speedup vs baseline: 7.3854x; 1.0782x over previous
import jax
import jax.numpy as jnp
from jax import lax
from jax.experimental import pallas as pl
from jax.experimental.pallas import tpu as pltpu
from jax.experimental.pallas import tpu_sc as plsc

VOCAB = 100000
D_MODEL = 1024
BATCH = 4
SEQ = 2048
N_TOKENS = 20

_NC = 2
_NS = 16
_NW = _NC * _NS
_ROWS = BATCH * SEQ
_RPW = _ROWS // _NW
_CH = 32
_NCHUNK = _RPW // _CH
_WPB = SEQ // _RPW
_PSTRIDE = 32
_NBUF = 3


def _body(idx_hbm, wte_hbm, learned_hbm, pidx_hbm, out_hbm,
          idx_v, rows0_v, rows1_v, rows2_v, learned_v, pidx_v,
          gsem0, gsem1, gsem2, ssem0, ssem1, ssem2, psem):
    wid = lax.axis_index("s") * _NC + lax.axis_index("c")
    base = wid * _RPW
    owns_prompt = wid % _WPB == 0
    b = wid // _WPB

    @pl.when(owns_prompt)
    def _():
        pltpu.async_copy(pidx_hbm.at[pl.ds(b * _PSTRIDE, N_TOKENS)],
                         pidx_v, psem)
        pltpu.async_copy(learned_hbm, learned_v, psem)

    pltpu.sync_copy(idx_hbm.at[pl.ds(base, _RPW)], idx_v)

    bufs = (rows0_v, rows1_v, rows2_v)
    gsems = (gsem0, gsem1, gsem2)
    ssems = (ssem0, ssem1, ssem2)

    def gather(j):
        return pltpu.async_copy(
            wte_hbm.at[idx_v.at[pl.ds(j * _CH, _CH)]], bufs[j % _NBUF],
            gsems[j % _NBUF])

    def scatter(j):
        return pltpu.async_copy(
            bufs[j % _NBUF], out_hbm.at[pl.ds(base + j * _CH, _CH)],
            ssems[j % _NBUF])

    g = {j: gather(j) for j in range(_NBUF)}
    s = {}
    prompt = {}
    for j in range(_NCHUNK):
        g[j].wait()
        s[j] = scatter(j)
        if j + _NBUF < _NCHUNK:
            s[j].wait()
            g[j + _NBUF] = gather(j + _NBUF)
        if j == 0:
            if _NBUF >= _NCHUNK:
                s[0].wait()

            @pl.when(owns_prompt)
            def _():
                pltpu.make_async_copy(pidx_hbm.at[pl.ds(0, N_TOKENS)],
                                      pidx_v, psem).wait()
                pltpu.make_async_copy(learned_hbm, learned_v, psem).wait()
                prompt[0] = pltpu.async_copy(
                    learned_v, out_hbm.at[pidx_v], psem)
    for j in range(_NCHUNK - _NBUF, _NCHUNK):
        s[j].wait()

    @pl.when(owns_prompt)
    def _():
        prompt[0].wait()


@jax.jit
def _soft_embedding(tokens, wte_weight, learned_embedding):
    idx = tokens.reshape(_ROWS)
    t = jnp.arange(_PSTRIDE, dtype=jnp.int32) % N_TOKENS
    pidx = (jnp.arange(BATCH, dtype=jnp.int32)[:, None] * SEQ + 1 + t[None, :]
            ).reshape(BATCH * _PSTRIDE)
    mesh = plsc.VectorSubcoreMesh(core_axis_name="c", subcore_axis_name="s")
    out = pl.kernel(
        _body,
        out_type=jax.ShapeDtypeStruct((_ROWS, D_MODEL), jnp.float32),
        mesh=mesh,
        scratch_types=[
            pltpu.VMEM((_RPW,), jnp.int32),
            pltpu.VMEM((_CH, D_MODEL), jnp.float32),
            pltpu.VMEM((_CH, D_MODEL), jnp.float32),
            pltpu.VMEM((_CH, D_MODEL), jnp.float32),
            pltpu.VMEM((N_TOKENS, D_MODEL), jnp.float32),
            pltpu.VMEM((N_TOKENS,), jnp.int32),
            pltpu.SemaphoreType.DMA,
            pltpu.SemaphoreType.DMA,
            pltpu.SemaphoreType.DMA,
            pltpu.SemaphoreType.DMA,
            pltpu.SemaphoreType.DMA,
            pltpu.SemaphoreType.DMA,
            pltpu.SemaphoreType.DMA,
        ],
    )(idx, wte_weight, learned_embedding, pidx)
    return out.reshape(BATCH, SEQ, D_MODEL)


def kernel(tokens, wte_weight, learned_embedding):
    return _soft_embedding(tokens, wte_weight, learned_embedding)
